# baseline (speedup 1.0000x reference)
import jax
import jax.numpy as jnp
from jax import lax
from jax.experimental import pallas as pl
from jax.experimental.pallas import tpu as pltpu
from jax.experimental.pallas import tpu_sc as plsc

B = 1024
D = 300
DP = 384
H = 1000
HP = 1024
K = 20
RHO = 1.0 - 0.85
EPS = 1e-6
NEG = -3e38

NC = 2
NS = 16
L = 16
NW = NC * NS
NG = B // (NW * L)
NCH = 8
CW = B // NCH
CMP = 128


def _cos_matrix(v):
    inv = 1.0 / jnp.maximum(jnp.sqrt(jnp.sum(v * v, axis=1, keepdims=True)), EPS)
    g = lax.dot_general(v, v, (((1,), (1,)), ((), ())),
                        preferred_element_type=jnp.float32)
    rowid = lax.broadcasted_iota(jnp.int32, (B, B), 0)
    colid = lax.broadcasted_iota(jnp.int32, (B, B), 1)
    m = jnp.where(rowid == colid, -10.0, g * inv * inv.T)
    cm = jnp.max(m.reshape(B, NCH, CW), axis=2)
    cm = jnp.concatenate(
        [cm, jnp.full((B, CMP - NCH), NEG, jnp.float32)], axis=1)
    return m, cm


def _tc(x_ref, y_ref, w1_ref, b1_ref, w2_ref, b2_ref,
        out_ref, h_ref, loss_ref, my_ref, cmy_ref, mh_ref, cmh_ref):
    x = x_ref[...]
    y = y_ref[...]

    l1 = lax.dot_general(x, w1_ref[...], (((1,), (1,)), ((), ())),
                         preferred_element_type=jnp.float32)
    h = jnp.clip(l1 + b1_ref[...], 0.0, 1.0)
    h_ref[...] = h

    out = lax.dot_general(h, w2_ref[...], (((1,), (1,)), ((), ())),
                          preferred_element_type=jnp.float32) + b2_ref[...]
    out_ref[...] = out

    loss_ref[0, 0] = jnp.sum((out - y) ** 2) / (B * D)
    loss_ref[0, 1] = jnp.sum(h * (1.0 - h)) / (B * H)
    colmean = jnp.sum(h, axis=0, keepdims=True) / B
    temp = jnp.maximum(colmean - RHO, 0.0)
    loss_ref[0, 2] = jnp.sum(temp * temp) / H

    my_ref[...], cmy_ref[...] = _cos_matrix(y)
    mh_ref[...], cmh_ref[...] = _cos_matrix(h)


def _extract_step(mb, lbase, cms, v):
    cidx = [jnp.full((L,), NCH, jnp.int32) for _ in range(2)]
    nmax = [jnp.full((L,), NEG, jnp.float32) for _ in range(2)]
    for c in range(NCH):
        cidx[c % 2] = jnp.minimum(cidx[c % 2], jnp.where(cms[c] == v, c, NCH))
        nmax[c % 2] = jnp.maximum(nmax[c % 2], jnp.where(cms[c] < v, cms[c], NEG))
    ci = jnp.minimum(cidx[0], cidx[1])
    nm = jnp.maximum(nmax[0], nmax[1])
    base = lbase + ci * CW
    macc = [jnp.full((L,), NEG, jnp.float32) for _ in range(4)]
    for p in range(CW):
        x = plsc.load_gather(mb, [base + p])
        macc[p % 4] = jnp.maximum(macc[p % 4], jnp.where(x < v, x, NEG))
    m = jnp.maximum(jnp.maximum(macc[0], macc[1]), jnp.maximum(macc[2], macc[3]))
    cms = [jnp.where(ci == c, m, cms[c]) for c in range(NCH)]
    return cms, jnp.maximum(nm, m)


def _cm_load(cb, cbase):
    return [plsc.load_gather(cb, [cbase + c]) for c in range(NCH)]


def _sc(my_hbm, cmy_hbm, mh_hbm, cmh_hbm, out_hbm,
        mby, mbh, cby, cbh, av, sem):
    w = lax.axis_index("c") * NS + lax.axis_index("s")
    lane = lax.iota(jnp.int32, L)
    lbase = lane * B
    cbase = lane * CMP
    acc = jnp.zeros((L,), jnp.float32)

    for g in range(NG):
        rb = (w * NG + g) * L
        cps = [
            pltpu.async_copy(my_hbm.at[pl.ds(rb * B, L * B)], mby, sem),
            pltpu.async_copy(mh_hbm.at[pl.ds(rb * B, L * B)], mbh, sem),
            pltpu.async_copy(cmy_hbm.at[pl.ds(rb * CMP, L * CMP)], cby, sem),
            pltpu.async_copy(cmh_hbm.at[pl.ds(rb * CMP, L * CMP)], cbh, sem),
        ]
        for cp in cps:
            cp.wait()

        cy = _cm_load(cby, cbase)
        ch = _cm_load(cbh, cbase)
        vy = cy[0]
        vh = ch[0]
        for c in range(1, NCH):
            vy = jnp.maximum(vy, cy[c])
            vh = jnp.maximum(vh, ch[c])
        acc = acc + jnp.abs(vy - vh)

        def step(_, carry):
            cy = list(carry[0:NCH])
            ch = list(carry[NCH:2 * NCH])
            vy, vh, acc = carry[2 * NCH:]
            cy, vy = _extract_step(mby, lbase, cy, vy)
            ch, vh = _extract_step(mbh, lbase, ch, vh)
            acc = acc + jnp.abs(vy - vh)
            return tuple(cy) + tuple(ch) + (vy, vh, acc)

        carry = lax.fori_loop(0, K - 1, step,
                              tuple(cy) + tuple(ch) + (vy, vh, acc))
        acc = carry[-1]

    av[...] = acc
    pltpu.sync_copy(av, out_hbm.at[pl.ds(w * L, L)])


def _sc_mesh():
    return plsc.VectorSubcoreMesh(core_axis_name="c", subcore_axis_name="s",
                                  num_cores=NC, num_subcores=NS)


@jax.jit
def kernel(batch_x, batch_y, W1, b1, W2, b2):
    xp = jnp.pad(batch_x, ((0, 0), (0, DP - D)))
    yp = jnp.pad(batch_y, ((0, 0), (0, DP - D)))
    w1p = jnp.pad(W1, ((0, HP - H), (0, DP - D)))
    b1p = jnp.pad(b1, (0, HP - H)).reshape(1, HP)
    w2p = jnp.pad(W2, ((0, DP - D), (0, HP - H)))
    b2p = jnp.pad(b2, (0, DP - D)).reshape(1, DP)

    out_p, h_p, loss, my, cmy, mh, cmh = pl.pallas_call(
        _tc,
        out_shape=[
            jax.ShapeDtypeStruct((B, DP), jnp.float32),
            jax.ShapeDtypeStruct((B, HP), jnp.float32),
            jax.ShapeDtypeStruct((1, 8), jnp.float32),
            jax.ShapeDtypeStruct((B, B), jnp.float32),
            jax.ShapeDtypeStruct((B, CMP), jnp.float32),
            jax.ShapeDtypeStruct((B, B), jnp.float32),
            jax.ShapeDtypeStruct((B, CMP), jnp.float32),
        ],
        out_specs=[
            pl.BlockSpec(memory_space=pltpu.VMEM),
            pl.BlockSpec(memory_space=pltpu.VMEM),
            pl.BlockSpec(memory_space=pltpu.SMEM),
            pl.BlockSpec(memory_space=pltpu.VMEM),
            pl.BlockSpec(memory_space=pltpu.VMEM),
            pl.BlockSpec(memory_space=pltpu.VMEM),
            pl.BlockSpec(memory_space=pltpu.VMEM),
        ],
    )(xp, yp, w1p, b1p, w2p, b2p)

    partial = pl.kernel(
        _sc,
        out_type=jax.ShapeDtypeStruct((NW * L,), jnp.float32),
        mesh=_sc_mesh(),
        compiler_params=pltpu.CompilerParams(needs_layout_passes=False),
        scratch_types=[
            pltpu.VMEM((L * B,), jnp.float32),
            pltpu.VMEM((L * B,), jnp.float32),
            pltpu.VMEM((L * CMP,), jnp.float32),
            pltpu.VMEM((L * CMP,), jnp.float32),
            pltpu.VMEM((L,), jnp.float32),
            pltpu.SemaphoreType.DMA,
        ],
    )(my.reshape(B * B), cmy.reshape(B * CMP),
      mh.reshape(B * B), cmh.reshape(B * CMP))

    out = out_p[:, :D]
    h = h_p[:, :H]
    recon = loss[0, 0]
    psl = loss[0, 1]
    asl = loss[0, 2]
    local = jnp.sum(partial) / (B * K)
    total = recon + psl + asl + local
    return (out, h, total, recon, psl, asl, local)


# speedup vs baseline: 1.7895x; 1.7895x over previous
import jax
import jax.numpy as jnp
from jax import lax
from jax.experimental import pallas as pl
from jax.experimental.pallas import tpu as pltpu
from jax.experimental.pallas import tpu_sc as plsc

B = 1024
D = 300
DP = 384
H = 1000
HP = 1024
K = 20
KP = 32
RHO = 1.0 - 0.85
EPS = 1e-6
NEG = -3e38

NC = 2
NS = 16
L = 16
NW = NC * NS
NG = B // (NW * L)
NCH = 16
CW = B // NCH
CMP = 128


def _cos_matrix(v):
    inv = 1.0 / jnp.maximum(jnp.sqrt(jnp.sum(v * v, axis=1, keepdims=True)), EPS)
    g = lax.dot_general(v, v, (((1,), (1,)), ((), ())),
                        preferred_element_type=jnp.float32)
    rowid = lax.broadcasted_iota(jnp.int32, (B, B), 0)
    colid = lax.broadcasted_iota(jnp.int32, (B, B), 1)
    return jnp.where(rowid == colid, -10.0, g * inv * inv.T)


def _tc_a(y_ref, my_ref, cmy_ref):
    m = _cos_matrix(y_ref[...])
    my_ref[...] = m
    cm = jnp.max(m.reshape(B, NCH, CW), axis=2)
    cmy_ref[...] = jnp.concatenate(
        [cm, jnp.full((B, CMP - NCH), NEG, jnp.float32)], axis=1)


def _tc_b(x_ref, y_ref, w1_ref, b1_ref, w2_ref, b2_ref,
          out_ref, h_ref, loss_ref, th_ref, mh_ref):
    x = x_ref[...]
    y = y_ref[...]

    l1 = lax.dot_general(x, w1_ref[...], (((1,), (1,)), ((), ())),
                         preferred_element_type=jnp.float32)
    h = jnp.clip(l1 + b1_ref[...], 0.0, 1.0)
    h_ref[...] = h

    out = lax.dot_general(h, w2_ref[...], (((1,), (1,)), ((), ())),
                          preferred_element_type=jnp.float32) + b2_ref[...]
    out_ref[...] = out

    loss_ref[0, 0] = jnp.sum((out - y) ** 2) / (B * D)
    loss_ref[0, 1] = jnp.sum(h * (1.0 - h)) / (B * H)
    colmean = jnp.sum(h, axis=0, keepdims=True) / B
    temp = jnp.maximum(colmean - RHO, 0.0)
    loss_ref[0, 2] = jnp.sum(temp * temp) / H

    mh0 = _cos_matrix(h)
    mh_ref[...] = mh0
    vh = jnp.max(mh0, axis=1, keepdims=True)
    cols = [vh]
    for _ in range(K - 1):
        mh = mh_ref[...]
        vh = jnp.max(jnp.where(mh < vh, mh, NEG), axis=1, keepdims=True)
        cols.append(vh)
    cols.append(jnp.full((B, KP - K), 0.0, jnp.float32))
    th_ref[...] = jnp.concatenate(cols, axis=1)


def _tc_c(vy_ref, th_ref, loss_ref):
    vy = jnp.transpose(vy_ref[...], (0, 2, 1)).reshape(B, K)
    th = th_ref[...][:, :K]
    loss_ref[0, 0] = jnp.sum(jnp.abs(vy - th)) / (B * K)


def _extract_step(mb, cb, lbase, cbase, v):
    cidx = [jnp.full((L,), NCH, jnp.int32) for _ in range(2)]
    nmax = [jnp.full((L,), NEG, jnp.float32) for _ in range(2)]
    for c in range(NCH):
        cmc = plsc.load_gather(cb, [cbase + c])
        cidx[c % 2] = jnp.minimum(cidx[c % 2], jnp.where(cmc == v, c, NCH))
        nmax[c % 2] = jnp.maximum(nmax[c % 2], jnp.where(cmc < v, cmc, NEG))
    ci = jnp.minimum(cidx[0], cidx[1])
    nm = jnp.maximum(nmax[0], nmax[1])
    base = lbase + ci * CW
    macc = [jnp.full((L,), NEG, jnp.float32) for _ in range(4)]
    for p in range(CW):
        x = plsc.load_gather(mb, [base + p])
        macc[p % 4] = jnp.maximum(macc[p % 4], jnp.where(x < v, x, NEG))
    m = jnp.maximum(jnp.maximum(macc[0], macc[1]), jnp.maximum(macc[2], macc[3]))
    plsc.store_scatter(cb, [cbase + ci], m)
    return jnp.maximum(nm, m)


def _cb_init(cb, cbase):
    v = plsc.load_gather(cb, [cbase])
    for c in range(1, NCH):
        v = jnp.maximum(v, plsc.load_gather(cb, [cbase + c]))
    return v


def _sc_y(my_hbm, cmy_hbm, vals_hbm, mb, cb, vv):
    w = lax.axis_index("c") * NS + lax.axis_index("s")
    lane = lax.iota(jnp.int32, L)
    lbase = lane * B
    cbase = lane * CMP
    for g in range(NG):
        gi = w * NG + g
        rb = gi * L
        pltpu.sync_copy(my_hbm.at[pl.ds(rb * B, L * B)], mb)
        pltpu.sync_copy(cmy_hbm.at[pl.ds(rb * CMP, L * CMP)], cb)
        v = _cb_init(cb, cbase)

        def step(k, v):
            plsc.store_scatter(vv, [k * L + lane], v)
            return _extract_step(mb, cb, lbase, cbase, v)

        v = lax.fori_loop(0, K - 1, step, v)
        plsc.store_scatter(vv, [(K - 1) * L + lane], v)
        pltpu.sync_copy(vv, vals_hbm.at[pl.ds(gi * K * L, K * L)])


def _sc_mesh():
    return plsc.VectorSubcoreMesh(core_axis_name="c", subcore_axis_name="s",
                                  num_cores=NC, num_subcores=NS)


@jax.jit
def kernel(batch_x, batch_y, W1, b1, W2, b2):
    xp = jnp.pad(batch_x, ((0, 0), (0, DP - D)))
    yp = jnp.pad(batch_y, ((0, 0), (0, DP - D)))
    w1p = jnp.pad(W1, ((0, HP - H), (0, DP - D)))
    b1p = jnp.pad(b1, (0, HP - H)).reshape(1, HP)
    w2p = jnp.pad(W2, ((0, DP - D), (0, HP - H)))
    b2p = jnp.pad(b2, (0, DP - D)).reshape(1, DP)

    my, cmy = pl.pallas_call(
        _tc_a,
        out_shape=[
            jax.ShapeDtypeStruct((B, B), jnp.float32),
            jax.ShapeDtypeStruct((B, CMP), jnp.float32),
        ],
    )(yp)

    vals_y = pl.kernel(
        _sc_y,
        out_type=jax.ShapeDtypeStruct((NW * NG * K * L,), jnp.float32),
        mesh=_sc_mesh(),
        compiler_params=pltpu.CompilerParams(needs_layout_passes=False),
        scratch_types=[
            pltpu.VMEM((L * B,), jnp.float32),
            pltpu.VMEM((L * CMP,), jnp.float32),
            pltpu.VMEM((K * L,), jnp.float32),
        ],
    )(my.reshape(B * B), cmy.reshape(B * CMP))

    out_p, h_p, loss, th = pl.pallas_call(
        _tc_b,
        out_shape=[
            jax.ShapeDtypeStruct((B, DP), jnp.float32),
            jax.ShapeDtypeStruct((B, HP), jnp.float32),
            jax.ShapeDtypeStruct((1, 8), jnp.float32),
            jax.ShapeDtypeStruct((B, KP), jnp.float32),
        ],
        out_specs=[
            pl.BlockSpec(memory_space=pltpu.VMEM),
            pl.BlockSpec(memory_space=pltpu.VMEM),
            pl.BlockSpec(memory_space=pltpu.SMEM),
            pl.BlockSpec(memory_space=pltpu.VMEM),
        ],
        scratch_shapes=[pltpu.VMEM((B, B), jnp.float32)],
    )(xp, yp, w1p, b1p, w2p, b2p)

    lloss = pl.pallas_call(
        _tc_c,
        out_shape=jax.ShapeDtypeStruct((1, 1), jnp.float32),
        out_specs=pl.BlockSpec(memory_space=pltpu.SMEM),
    )(vals_y.reshape(NW * NG, K, L), th)

    out = out_p[:, :D]
    h = h_p[:, :H]
    recon = loss[0, 0]
    psl = loss[0, 1]
    asl = loss[0, 2]
    local = lloss[0, 0]
    total = recon + psl + asl + local
    return (out, h, total, recon, psl, asl, local)
